# Initial kernel scaffold; baseline (speedup 1.0000x reference)
import jax
import jax.numpy as jnp
from jax.experimental import pallas as pl


def kernel(x, edge_index, edge_weight, index, W1, b1, W2, b2):
    raise NotImplementedError("write your pallas kernel here")


# speedup vs baseline: 7.5281x; 7.5281x over previous
import functools

import jax
import jax.numpy as jnp
from jax import lax
from jax.experimental import pallas as pl
from jax.experimental.pallas import tpu as pltpu
from jax.experimental.pallas import tpu_sc as plsc

N = 10000
E = 320000
D = 128
B = 4096

NC = 2
NS = 16
NW = NC * NS
L = 16

C = 128
NCH = (E // NW + C - 1) // C
EPT = NCH * C
EPAD = NW * EPT

NP = 10240
ROWS_PER_TILE = NP // NS

BR = 1024
PAIRS_PER_TILE = B // NW


def _sc_mesh():
    return plsc.VectorSubcoreMesh(
        core_axis_name="c", subcore_axis_name="s", num_cores=NC,
        num_subcores=NS)


_SC_PARAMS = pltpu.CompilerParams(needs_layout_passes=False)


def _deg_body(src_hbm, dst_hbm, w_hbm, out_hbm,
              sidx, didx, wch, deg_s, deg_d):
    c = lax.axis_index("c")
    s = lax.axis_index("s")
    wid = c * NS + s

    def zero_body(i, _):
        z = jnp.zeros((L,), jnp.float32)
        deg_s[pl.ds(i * L, L)] = z
        deg_d[pl.ds(i * L, L)] = z
        return _

    lax.fori_loop(0, NP // L, zero_body, None)

    def chunk_body(g, _):
        pltpu.sync_copy(src_hbm.at[wid, g], sidx)
        pltpu.sync_copy(dst_hbm.at[wid, g], didx)
        pltpu.sync_copy(w_hbm.at[wid, g], wch)
        for j in range(C // L):
            sl = pl.ds(j * L, L)
            wv = wch[sl]
            plsc.addupdate_scatter(deg_s, [sidx[sl]], wv)
            plsc.addupdate_scatter(deg_d, [didx[sl]], wv)
        return _

    lax.fori_loop(0, NCH, chunk_body, None)
    pltpu.sync_copy(deg_s, out_hbm.at[wid, 0])
    pltpu.sync_copy(deg_d, out_hbm.at[wid, 1])


def _sc_degrees(srcp, dstp, wp):
    return pl.kernel(
        _deg_body,
        out_type=jax.ShapeDtypeStruct((NW, 2, NP), jnp.float32),
        mesh=_sc_mesh(),
        compiler_params=_SC_PARAMS,
        scratch_types=[
            pltpu.VMEM((C,), jnp.int32),
            pltpu.VMEM((C,), jnp.int32),
            pltpu.VMEM((C,), jnp.float32),
            pltpu.VMEM((NP,), jnp.float32),
            pltpu.VMEM((NP,), jnp.float32),
        ],
    )(srcp, dstp, wp)


def _rsqrt_body(degp_ref, out_ref):
    d = jnp.sum(degp_ref[...], axis=0)
    out_ref[...] = lax.rsqrt(jnp.maximum(d, 1e-6))


def _tc_rsqrt(degp):
    return pl.pallas_call(
        _rsqrt_body,
        out_shape=jax.ShapeDtypeStruct((2, NP), jnp.float32),
    )(degp)


def _mm_scale_body(x_ref, w_ref, rs_ref, out_ref):
    h = jnp.dot(x_ref[...], w_ref[...], preferred_element_type=jnp.float32)
    out_ref[...] = h * rs_ref[...]


def _tc_mm_scale(xp, W, rs):
    grid = NP // BR
    return pl.pallas_call(
        _mm_scale_body,
        grid=(grid,),
        in_specs=[
            pl.BlockSpec((BR, D), lambda i: (i, 0)),
            pl.BlockSpec((D, D), lambda i: (0, 0)),
            pl.BlockSpec((BR, 1), lambda i: (i, 0)),
        ],
        out_specs=pl.BlockSpec((BR, D), lambda i: (i, 0)),
        out_shape=jax.ShapeDtypeStruct((NP, D), jnp.float32),
    )(xp, W, rs)


def _edge_body(g_hbm, src_hbm, dst_hbm, w_hbm, out_hbm,
               sidx, didx, wch, rows, acc, sem):
    c = lax.axis_index("c")
    s = lax.axis_index("s")
    wid = c * NS + s

    def zbuf_body(r, _):
        z = jnp.zeros((L,), jnp.float32)
        for k in range(D // L):
            rows[r, pl.ds(k * L, L)] = z
        return _

    lax.fori_loop(0, C, zbuf_body, None)
    for r in range(ROWS_PER_TILE // C):
        base = s * ROWS_PER_TILE + r * C
        pltpu.sync_copy(rows, acc.at[pl.ds(base, C)])
    plsc.subcore_barrier()

    def chunk_body(g, _):
        pltpu.sync_copy(src_hbm.at[wid, g], sidx)
        pltpu.sync_copy(dst_hbm.at[wid, g], didx)
        pltpu.sync_copy(w_hbm.at[wid, g], wch)
        pltpu.async_copy(g_hbm.at[sidx], rows, sem).wait()

        def scale_body(e16, _):
            wv = wch[pl.ds(e16 * L, L)]
            for j in range(L):
                e = e16 * L + j
                we = wv[j]
                for k in range(D // L):
                    sl = pl.ds(k * L, L)
                    rows[e, sl] = rows[e, sl] * we
            return _

        lax.fori_loop(0, C // L, scale_body, None)
        pltpu.sync_copy(rows, acc.at[didx], add=True)
        return _

    lax.fori_loop(0, NCH, chunk_body, None)
    plsc.subcore_barrier()
    base = s * ROWS_PER_TILE
    pltpu.sync_copy(acc.at[pl.ds(base, ROWS_PER_TILE)],
                    out_hbm.at[c, pl.ds(base, ROWS_PER_TILE)])


def _sc_edge_pass(g, srcp, dstp, wp):
    return pl.kernel(
        _edge_body,
        out_type=jax.ShapeDtypeStruct((NC, NP, D), jnp.float32),
        mesh=_sc_mesh(),
        compiler_params=_SC_PARAMS,
        scratch_types=[
            pltpu.VMEM((C,), jnp.int32),
            pltpu.VMEM((C,), jnp.int32),
            pltpu.VMEM((C,), jnp.float32),
            pltpu.VMEM((C, D), jnp.float32),
            pltpu.VMEM_SHARED((NP, D), jnp.float32),
            pltpu.SemaphoreType.DMA,
        ],
    )(g, srcp, dstp, wp)


def _mid_body(a_ref, b_ref, rd_ref, rs_ref, b1_ref, w_ref, out_ref):
    h = (a_ref[...] + b_ref[...]) * rd_ref[...] + b1_ref[...]
    h = jnp.maximum(h, 0.0)
    g2 = jnp.dot(h, w_ref[...], preferred_element_type=jnp.float32)
    out_ref[...] = g2 * rs_ref[...]


def _tc_mid(accA, accB, rd, rs, b1, W2):
    grid = NP // BR
    return pl.pallas_call(
        _mid_body,
        grid=(grid,),
        in_specs=[
            pl.BlockSpec((BR, D), lambda i: (i, 0)),
            pl.BlockSpec((BR, D), lambda i: (i, 0)),
            pl.BlockSpec((BR, 1), lambda i: (i, 0)),
            pl.BlockSpec((BR, 1), lambda i: (i, 0)),
            pl.BlockSpec((D,), lambda i: (0,)),
            pl.BlockSpec((D, D), lambda i: (0, 0)),
        ],
        out_specs=pl.BlockSpec((BR, D), lambda i: (i, 0)),
        out_shape=jax.ShapeDtypeStruct((NP, D), jnp.float32),
    )(accA, accB, rd, rs, b1, W2)


def _post_body(a_ref, b_ref, rd_ref, b2_ref, out_ref):
    out_ref[...] = (a_ref[...] + b_ref[...]) * rd_ref[...] + b2_ref[...]


def _tc_post(accA, accB, rd, b2):
    grid = NP // BR
    return pl.pallas_call(
        _post_body,
        grid=(grid,),
        in_specs=[
            pl.BlockSpec((BR, D), lambda i: (i, 0)),
            pl.BlockSpec((BR, D), lambda i: (i, 0)),
            pl.BlockSpec((BR, 1), lambda i: (i, 0)),
            pl.BlockSpec((D,), lambda i: (0,)),
        ],
        out_specs=pl.BlockSpec((BR, D), lambda i: (i, 0)),
        out_shape=jax.ShapeDtypeStruct((NP, D), jnp.float32),
    )(accA, accB, rd, b2)


def _decode_body(emb_hbm, idx_hbm, out_hbm,
                 sidx, didx, srows, drows, olocal, sem):
    c = lax.axis_index("c")
    s = lax.axis_index("s")
    wid = c * NS + s
    P = PAIRS_PER_TILE

    pltpu.sync_copy(idx_hbm.at[0, wid], sidx)
    pltpu.sync_copy(idx_hbm.at[1, wid], didx)
    pltpu.async_copy(emb_hbm.at[sidx], srows, sem).wait()
    pltpu.async_copy(emb_hbm.at[didx], drows, sem).wait()

    for pg in range(P // L):
        row_idx = pg * L + jnp.arange(L, dtype=jnp.int32)

        def dim_body(d, acc):
            col = jnp.full((L,), d, dtype=jnp.int32)
            sv = plsc.load_gather(srows, [row_idx, col])
            dv = plsc.load_gather(drows, [row_idx, col])
            return acc + sv * dv

        out16 = lax.fori_loop(0, D, dim_body,
                              jnp.zeros((L,), jnp.float32))
        olocal[pl.ds(pg * L, L)] = out16
    pltpu.sync_copy(olocal, out_hbm.at[pl.ds(wid * P, P)])


def _sc_decode(emb, idxp):
    P = PAIRS_PER_TILE
    return pl.kernel(
        _decode_body,
        out_type=jax.ShapeDtypeStruct((B,), jnp.float32),
        mesh=_sc_mesh(),
        compiler_params=_SC_PARAMS,
        scratch_types=[
            pltpu.VMEM((P,), jnp.int32),
            pltpu.VMEM((P,), jnp.int32),
            pltpu.VMEM((P, D), jnp.float32),
            pltpu.VMEM((P, D), jnp.float32),
            pltpu.VMEM((P,), jnp.float32),
            pltpu.SemaphoreType.DMA,
        ],
    )(emb, idxp)


@jax.jit
def kernel(x, edge_index, edge_weight, index, W1, b1, W2, b2):
    src = edge_index[0].astype(jnp.int32)
    dst = edge_index[1].astype(jnp.int32)
    w = edge_weight.astype(jnp.float32)

    pad = EPAD - E
    srcp = jnp.pad(src, (0, pad)).reshape(NW, NCH, C)
    dstp = jnp.pad(dst, (0, pad)).reshape(NW, NCH, C)
    wp = jnp.pad(w, (0, pad)).reshape(NW, NCH, C)
    xp = jnp.pad(x, ((0, NP - N), (0, 0)))
    idxp = index.astype(jnp.int32).reshape(2, NW, PAIRS_PER_TILE)

    degp = _sc_degrees(srcp, dstp, wp)
    r = _tc_rsqrt(degp)
    rs = r[0].reshape(NP, 1)
    rd = r[1].reshape(NP, 1)

    g1 = _tc_mm_scale(xp, W1, rs)
    acc1 = _sc_edge_pass(g1, srcp, dstp, wp)
    g2 = _tc_mid(acc1[0], acc1[1], rd, rs, b1, W2)
    acc2 = _sc_edge_pass(g2, srcp, dstp, wp)
    emb = _tc_post(acc2[0], acc2[1], rd, b2)
    out = _sc_decode(emb, idxp)
    return out
